# Initial kernel scaffold; baseline (speedup 1.0000x reference)
import jax
import jax.numpy as jnp
from jax.experimental import pallas as pl


def kernel(e_conv1, e_conv2, e_conv3, e_fc1_w, e_fc1_b, e_bn1_g, e_bn1_b, e_mean_w, e_mean_b, e_fc2_w, e_fc2_b, e_bn2_g, e_bn2_b, e_lv_w, e_lv_b, d_fc1_w, d_fc1_b, d_bn1_g, d_bn1_b, d_dc1, d_dc2, d_dc3, d_dc4, x, eps):
    raise NotImplementedError("write your pallas kernel here")


# speedup vs baseline: 97.6581x; 97.6581x over previous
import functools

import numpy as np
import jax
import jax.numpy as jnp
from jax.experimental import pallas as pl
from jax.experimental.pallas import tpu as pltpu

NC = 3
NGF = 8
NZ = 16
FEAT = NGF * 4 * 8 * 8
BN_EPS = 1e-5

F32 = jnp.float32


def _sel_conv(W_in, OW, k=5, stride=2, pad=2):
    kj = np.arange(k)[:, None, None]
    w = np.arange(W_in)[None, :, None]
    ow = np.arange(OW)[None, None, :]
    return (w == stride * ow + kj - pad).astype(np.float32)


def _sel_deconv(W_in, OW, k=5, stride=2, pad=2):
    kj = np.arange(k)[:, None, None]
    w = np.arange(W_in)[None, :, None]
    ow = np.arange(OW)[None, None, :]
    return (ow == stride * w + kj - pad).astype(np.float32)


_S1 = _sel_conv(64, 32)
_S2 = _sel_conv(32, 16)
_S3 = _sel_conv(16, 8)
_SD1 = _sel_deconv(8, 16)
_SD2 = _sel_deconv(16, 32)
_SD3 = _sel_deconv(32, 64)
_S4 = _sel_conv(64, 64, k=5, stride=1, pad=2)


def _enc_kernel(xe_ref, xo_ref, b1_ref, b2_ref, b3_ref, w12_ref, o_ref,
                xpe, xpo, xp2a, xp2b, xp3a, xp3b, *, tb):
    for ref, src in ((xpe, xe_ref), (xpo, xo_ref)):
        ref[:, :, 0:1, :] = jnp.zeros((tb, 3, 1, 64), F32)
        ref[:, :, 33:34, :] = jnp.zeros((tb, 3, 1, 64), F32)
        ref[:, :, 1:33, :] = src[...]

    acc1 = jnp.zeros((tb * 32, 256), F32)
    for ki in range(5):
        par = xpe if ki % 2 == 0 else xpo
        lhs = jnp.concatenate(
            [par[:, ic, pl.ds(ki // 2, 32), :] for ic in range(3)],
            axis=-1).reshape(tb * 32, 192)
        acc1 = acc1 + jnp.dot(lhs, b1_ref[ki], preferred_element_type=F32)

    v1 = acc1.reshape(tb, 32, 256)
    for ref, lo in ((xp2a, 0), (xp2b, 128)):
        ref[:, 0:2, :] = jnp.zeros((tb, 2, 128), F32)
        ref[:, 34:36, :] = jnp.zeros((tb, 2, 128), F32)
        ref[:, 2:34, :] = v1[:, :, lo:lo + 128]

    acc2 = jnp.zeros((tb * 16, 256), F32)
    for ki in range(5):
        lhs = jnp.concatenate(
            [xp2a[:, pl.ds(ki, 16, 2), :], xp2b[:, pl.ds(ki, 16, 2), :]],
            axis=-1).reshape(tb * 16, 256)
        acc2 = acc2 + jnp.dot(lhs, b2_ref[ki], preferred_element_type=F32)

    v2 = acc2.reshape(tb, 16, 256)
    for ref, lo in ((xp3a, 0), (xp3b, 128)):
        ref[:, 0:2, :] = jnp.zeros((tb, 2, 128), F32)
        ref[:, 18:20, :] = jnp.zeros((tb, 2, 128), F32)
        ref[:, 2:18, :] = v2[:, :, lo:lo + 128]

    acc3 = jnp.zeros((tb * 8, 256), F32)
    for ki in range(5):
        lhs = jnp.concatenate(
            [xp3a[:, pl.ds(ki, 8, 2), :], xp3b[:, pl.ds(ki, 8, 2), :]],
            axis=-1).reshape(tb * 8, 256)
        acc3 = acc3 + jnp.dot(lhs, b3_ref[ki], preferred_element_type=F32)

    t3 = acc3.reshape(tb, 8, 256)
    x12 = jnp.zeros((tb, 2 * NZ), F32)
    for h in range(8):
        x12 = x12 + jnp.dot(t3[:, h, :], w12_ref[h],
                            preferred_element_type=F32)
    o_ref[...] = x12


def _mid_kernel(x12_ref, eps_ref,
                b1_ref, g1_ref, be1_ref, wm_ref, bm_ref,
                b2_ref, g2_ref, be2_ref, wl_ref, bl_ref,
                wd_ref, bd_ref, gd_ref, bed_ref, o_ref):
    def bn(v, g, b):
        m = jnp.mean(v, axis=0, keepdims=True)
        var = jnp.mean(jnp.square(v - m), axis=0, keepdims=True)
        return (v - m) * jax.lax.rsqrt(var + BN_EPS) * g + b

    x12 = x12_ref[...]
    x1 = bn(x12[:, 0:NZ] + b1_ref[...], g1_ref[...], be1_ref[...])
    x2 = bn(x12[:, NZ:2 * NZ] + b2_ref[...], g2_ref[...], be2_ref[...])
    mu = jnp.dot(x1, wm_ref[...], preferred_element_type=F32) + bm_ref[...]
    logvar = jnp.dot(x2, wl_ref[...], preferred_element_type=F32) + bl_ref[...]
    z = eps_ref[...] * jnp.exp(0.5 * logvar) + mu
    h = jnp.dot(z, wd_ref[...], preferred_element_type=F32) + bd_ref[...]
    h = bn(h, gd_ref[...], bed_ref[...])
    o_ref[...] = jnp.maximum(h, 0.0)


def _dec_kernel(h_ref, d1_ref, d2_ref, d3_ref, c4_ref, o_ref,
                p1, p2, p3, aE, aO, *, tb):
    def deconv(lhs, h_in, n, d_ref, dst_ref, dst_interior):
        aE[:, 0:h_in + 2, 0:n] = jnp.zeros((tb, h_in + 2, n), F32)
        aO[:, 0:h_in + 2, 0:n] = jnp.zeros((tb, h_in + 2, n), F32)
        for ki in range(5):
            acc = jnp.dot(lhs, d_ref[ki], preferred_element_type=F32)
            acc = acc.reshape(tb, h_in, n)
            if ki % 2 == 0:
                aE[:, pl.ds(ki // 2, h_in), 0:n] += acc
            else:
                aO[:, pl.ds(ki // 2, h_in), 0:n] += acc
        for j in range(n // 128):
            lo = j * 128
            dst_ref[j, :, pl.ds(dst_interior + 0, h_in, 2), :] = \
                aE[:, 1:h_in + 1, lo:lo + 128]
            dst_ref[j, :, pl.ds(dst_interior + 1, h_in, 2), :] = \
                aO[:, 1:h_in + 1, lo:lo + 128]

    deconv(h_ref[...], 8, 512, d1_ref, p1, 0)
    lhs2 = jnp.concatenate([p1[j] for j in range(4)],
                           axis=-1).reshape(tb * 16, 512)
    deconv(lhs2, 16, 512, d2_ref, p2, 0)
    lhs3 = jnp.concatenate([p2[j] for j in range(4)],
                           axis=-1).reshape(tb * 32, 512)
    p3[:, :, 0:2, :] = jnp.zeros((2, tb, 2, 128), F32)
    p3[:, :, 66:68, :] = jnp.zeros((2, tb, 2, 128), F32)
    deconv(lhs3, 32, 256, d3_ref, p3, 2)

    acc4 = jnp.zeros((tb * 64, 384), F32)
    for ki in range(5):
        lhs = jnp.concatenate(
            [p3[0, :, pl.ds(ki, 64), :], p3[1, :, pl.ds(ki, 64), :]],
            axis=-1).reshape(tb * 64, 256)
        acc4 = acc4 + jnp.dot(lhs, c4_ref[ki], preferred_element_type=F32)
    t = acc4.reshape(tb, 64, 384)
    for c in range(3):
        o_ref[:, c, :, :] = jnp.tanh(t[:, :, c * 128:c * 128 + 64])


def kernel(e_conv1, e_conv2, e_conv3, e_fc1_w, e_fc1_b, e_bn1_g, e_bn1_b,
           e_mean_w, e_mean_b, e_fc2_w, e_fc2_b, e_bn2_g, e_bn2_b,
           e_lv_w, e_lv_b, d_fc1_w, d_fc1_b, d_bn1_g, d_bn1_b,
           d_dc1, d_dc2, d_dc3, d_dc4, x, eps):
    B = x.shape[0]
    f = lambda v: v.astype(F32)
    r = lambda v: v.astype(F32).reshape(1, -1)

    s1 = jnp.asarray(_S1)
    s2 = jnp.asarray(_S2)
    s3 = jnp.asarray(_S3)
    sd1 = jnp.asarray(_SD1)
    sd2 = jnp.asarray(_SD2)
    sd3 = jnp.asarray(_SD3)
    s4 = jnp.asarray(_S4)

    B1 = jnp.einsum('jwo,Oikj->kiwoO', s1, f(e_conv1)).reshape(5, 192, 256)
    B2 = jnp.einsum('jwo,Oikj->kwioO', s2, f(e_conv2)).reshape(5, 256, 256)
    B3 = jnp.einsum('jwo,Oikj->kwioO', s3, f(e_conv3)).reshape(5, 256, 256)

    def fcr(wmat):
        return f(wmat).reshape(NGF * 4, 8, 8, NZ).transpose(1, 2, 0, 3) \
                      .reshape(8, 8 * NGF * 4, NZ)
    W12 = jnp.concatenate([fcr(e_fc1_w), fcr(e_fc2_w)], axis=-1)

    D1 = jnp.einsum('jwo,iOkj->kwioO', sd1, f(d_dc1)).reshape(5, 256, 512)
    D2 = jnp.einsum('jwo,iOkj->kwioO', sd2, f(d_dc2)).reshape(5, 512, 512)
    D3 = jnp.einsum('jwo,iOkj->kwioO', sd3, f(d_dc3)).reshape(5, 512, 256)
    wc = jnp.flip(f(d_dc4), axis=(2, 3)).transpose(1, 0, 2, 3)
    C4 = jnp.einsum('jwo,Oikj->kwiOo', s4, wc)
    C4 = jnp.pad(C4, ((0, 0), (0, 0), (0, 0), (0, 0), (0, 64)))
    C4 = C4.reshape(5, 256, 384)

    def dr(v):
        return f(v).reshape(NGF * 4, 8, 8).transpose(1, 2, 0).reshape(1, FEAT)
    Wd = f(d_fc1_w).reshape(NZ, NGF * 4, 8, 8).transpose(0, 2, 3, 1) \
                   .reshape(NZ, FEAT)

    tb = 64 if B % 64 == 0 else B
    grid = B // tb
    x12 = pl.pallas_call(
        functools.partial(_enc_kernel, tb=tb),
        out_shape=jax.ShapeDtypeStruct((B, 2 * NZ), F32),
        grid=(grid,),
        in_specs=[
            pl.BlockSpec((tb, 3, 32, 64), lambda i: (i, 0, 0, 0)),
            pl.BlockSpec((tb, 3, 32, 64), lambda i: (i, 0, 0, 0)),
            pl.BlockSpec((5, 192, 256), lambda i: (0, 0, 0)),
            pl.BlockSpec((5, 256, 256), lambda i: (0, 0, 0)),
            pl.BlockSpec((5, 256, 256), lambda i: (0, 0, 0)),
            pl.BlockSpec((8, 256, 2 * NZ), lambda i: (0, 0, 0)),
        ],
        out_specs=pl.BlockSpec((tb, 2 * NZ), lambda i: (i, 0)),
        scratch_shapes=[
            pltpu.VMEM((tb, 3, 34, 64), F32),
            pltpu.VMEM((tb, 3, 34, 64), F32),
            pltpu.VMEM((tb, 36, 128), F32),
            pltpu.VMEM((tb, 36, 128), F32),
            pltpu.VMEM((tb, 20, 128), F32),
            pltpu.VMEM((tb, 20, 128), F32),
        ],
        compiler_params=pltpu.CompilerParams(
            dimension_semantics=("parallel",)),
    )(f(x)[:, :, 0::2, :], f(x)[:, :, 1::2, :], B1, B2, B3, W12)

    hmid = pl.pallas_call(
        _mid_kernel,
        out_shape=jax.ShapeDtypeStruct((B, FEAT), F32),
        in_specs=[
            pl.BlockSpec((B, 2 * NZ), lambda: (0, 0)),
            pl.BlockSpec((B, NZ), lambda: (0, 0)),
            pl.BlockSpec((1, NZ), lambda: (0, 0)),
            pl.BlockSpec((1, NZ), lambda: (0, 0)),
            pl.BlockSpec((1, NZ), lambda: (0, 0)),
            pl.BlockSpec((NZ, NZ), lambda: (0, 0)),
            pl.BlockSpec((1, NZ), lambda: (0, 0)),
            pl.BlockSpec((1, NZ), lambda: (0, 0)),
            pl.BlockSpec((1, NZ), lambda: (0, 0)),
            pl.BlockSpec((1, NZ), lambda: (0, 0)),
            pl.BlockSpec((NZ, NZ), lambda: (0, 0)),
            pl.BlockSpec((1, NZ), lambda: (0, 0)),
            pl.BlockSpec((NZ, FEAT), lambda: (0, 0)),
            pl.BlockSpec((1, FEAT), lambda: (0, 0)),
            pl.BlockSpec((1, FEAT), lambda: (0, 0)),
            pl.BlockSpec((1, FEAT), lambda: (0, 0)),
        ],
        out_specs=pl.BlockSpec((B, FEAT), lambda: (0, 0)),
    )(x12, f(eps),
      r(e_fc1_b), r(e_bn1_g), r(e_bn1_b), f(e_mean_w), r(e_mean_b),
      r(e_fc2_b), r(e_bn2_g), r(e_bn2_b), f(e_lv_w), r(e_lv_b),
      Wd, dr(d_fc1_b), dr(d_bn1_g), dr(d_bn1_b))

    hdec = hmid.reshape(B * 8, 256)

    out = pl.pallas_call(
        functools.partial(_dec_kernel, tb=tb),
        out_shape=jax.ShapeDtypeStruct((B, 3, 64, 64), F32),
        grid=(grid,),
        in_specs=[
            pl.BlockSpec((tb * 8, 256), lambda i: (i, 0)),
            pl.BlockSpec((5, 256, 512), lambda i: (0, 0, 0)),
            pl.BlockSpec((5, 512, 512), lambda i: (0, 0, 0)),
            pl.BlockSpec((5, 512, 256), lambda i: (0, 0, 0)),
            pl.BlockSpec((5, 256, 384), lambda i: (0, 0, 0)),
        ],
        out_specs=pl.BlockSpec((tb, 3, 64, 64), lambda i: (i, 0, 0, 0)),
        scratch_shapes=[
            pltpu.VMEM((4, tb, 16, 128), F32),
            pltpu.VMEM((4, tb, 32, 128), F32),
            pltpu.VMEM((2, tb, 68, 128), F32),
            pltpu.VMEM((tb, 34, 512), F32),
            pltpu.VMEM((tb, 34, 512), F32),
        ],
        compiler_params=pltpu.CompilerParams(
            dimension_semantics=("parallel",)),
    )(hdec, D1, D2, D3, C4)
    return out
